# Initial kernel scaffold; baseline (speedup 1.0000x reference)
import jax
import jax.numpy as jnp
from jax.experimental import pallas as pl


def kernel(pos_edge_index, neg_edge_index, x, train_pos_edge_index, W1, b1, W2, b2, W3, b3, W4, b4, Wl, bl):
    raise NotImplementedError("write your pallas kernel here")


# speedup vs baseline: 2.7591x; 2.7591x over previous
import functools

import jax
import jax.numpy as jnp
from jax import lax
from jax.experimental import pallas as pl
from jax.experimental.pallas import tpu as pltpu

N = 10000
D = 1024
H2 = 512

ROW_BLK = 400


def _head_body(y_ref, x_ref, wcat_ref, bcat_ref, wlt_ref, wlb_ref, bl_ref, out_ref):
    y = y_ref[...]
    g = jnp.maximum(jnp.dot(y, wcat_ref[...], preferred_element_type=jnp.float32)
                    + bcat_ref[...], 0.0)
    h = (jnp.dot(g, wlt_ref[...], preferred_element_type=jnp.float32)
         + jnp.dot(x_ref[...], wlb_ref[...], preferred_element_type=jnp.float32)
         + bl_ref[...])
    out_ref[...] = jnp.maximum(h, 0.0)


def _head_matmuls(y, x, wcat, bcat, wl_top, wl_bot, bl):
    grid = (N // ROW_BLK,)
    return pl.pallas_call(
        _head_body,
        grid=grid,
        in_specs=[
            pl.BlockSpec((ROW_BLK, D), lambda i: (i, 0)),
            pl.BlockSpec((ROW_BLK, D), lambda i: (i, 0)),
            pl.BlockSpec((D, D), lambda i: (0, 0)),
            pl.BlockSpec((1, D), lambda i: (0, 0)),
            pl.BlockSpec((D, H2), lambda i: (0, 0)),
            pl.BlockSpec((D, H2), lambda i: (0, 0)),
            pl.BlockSpec((1, H2), lambda i: (0, 0)),
        ],
        out_specs=pl.BlockSpec((ROW_BLK, H2), lambda i: (i, 0)),
        out_shape=jax.ShapeDtypeStruct((N, H2), jnp.float32),
    )(y, x, wcat, bcat.reshape(1, D), wl_top, wl_bot, bl.reshape(1, H2))


def kernel(pos_edge_index, neg_edge_index, x, train_pos_edge_index,
           W1, b1, W2, b2, W3, b3, W4, b4, Wl, bl):
    src = train_pos_edge_index[0]
    dst = train_pos_edge_index[1]
    deg = jnp.zeros((N,), jnp.float32).at[dst].add(1.0) + 1.0
    dinv = lax.rsqrt(deg)
    xs = dinv[:, None] * x
    z = jnp.zeros_like(x).at[dst].add(xs[src])
    y = dinv[:, None] * (z + xs)

    wcat = jnp.concatenate([W1, W2, W3, W4], axis=1)
    bcat = jnp.concatenate([b1, b2, b3, b4])
    h2 = _head_matmuls(y, x, wcat, bcat, Wl[:D], Wl[D:], bl)

    te = jnp.concatenate([pos_edge_index, neg_edge_index], axis=-1)
    return jnp.einsum('ef,ef->e', h2[te[1]], h2[te[0]])


# speedup vs baseline: 4.6775x; 1.6953x over previous
import functools

import jax
import jax.numpy as jnp
from jax import lax
from jax.experimental import pallas as pl
from jax.experimental.pallas import tpu as pltpu
from jax.experimental.pallas import tpu_sc as plsc

N = 10000
NPAD = 10240
D = 1024
H2 = 512
F = 128
C = D // F
NC, NS, L = 2, 16, 16
NW = NC * NS
E_TRAIN = 160000
E_PAD = 163840
E_SCORE = 100000
EB = 32
NB2 = 104
E2_PAD = 32 * EB * NB2

ROW_BLK = 640

_mesh = plsc.VectorSubcoreMesh(core_axis_name="c", subcore_axis_name="s")


NBH = E_PAD // (NC * NS * 128)
NROWS_T = NPAD // NS


@functools.partial(
    pl.kernel,
    out_type=jax.ShapeDtypeStruct((NC * NPAD, L), jnp.float32),
    mesh=_mesh,
    scratch_types=[
        pltpu.VMEM((NBH, 128), jnp.int32),
        pltpu.VMEM((128, L), jnp.float32),
        pltpu.VMEM((128, L), jnp.float32),
        pltpu.VMEM_SHARED((NPAD, L), jnp.float32),
    ],
)
def _hist_k(dst_hbm, out_hbm, didx_v, ones_v, zero_v, deg_sh):
    cid = lax.axis_index("c")
    tid = lax.axis_index("s")
    pltpu.sync_copy(dst_hbm.at[pl.ds((cid * NS + tid) * NBH, NBH)], didx_v)
    ones16 = jnp.ones((L,), jnp.float32)
    zero16 = jnp.zeros((L,), jnp.float32)

    def fill(i, _):
        ones_v[i, :] = ones16
        zero_v[i, :] = zero16
        return 0

    lax.fori_loop(0, 128, fill, 0)
    for r in range(NROWS_T // 128):
        pltpu.sync_copy(zero_v, deg_sh.at[pl.ds(tid * NROWS_T + r * 128, 128)])
    plsc.subcore_barrier()

    def body(b, _):
        pltpu.sync_copy(ones_v, deg_sh.at[didx_v.at[b]], add=True)
        return 0

    lax.fori_loop(0, NBH, body, 0)
    plsc.subcore_barrier()
    pltpu.sync_copy(deg_sh.at[pl.ds(tid * NROWS_T, NROWS_T)],
                    out_hbm.at[pl.ds(cid * NPAD + tid * NROWS_T, NROWS_T)])


def _prep_body(parts_ref, x_ref, xs_ref, dinv_ref):
    p = parts_ref[:, :, 0:1]
    deg = p[0] + p[1] + 1.0
    dinv = lax.rsqrt(deg)
    xsb = x_ref[...] * dinv
    for c in range(C):
        xs_ref[c] = xsb[:, c * F:(c + 1) * F]
    dinv_ref[...] = dinv


def _prep(parts, x):
    return pl.pallas_call(
        _prep_body,
        grid=(NPAD // ROW_BLK,),
        in_specs=[
            pl.BlockSpec((NC, ROW_BLK, L), lambda i: (0, i, 0)),
            pl.BlockSpec((ROW_BLK, D), lambda i: (i, 0)),
        ],
        out_specs=[
            pl.BlockSpec((C, ROW_BLK, F), lambda i: (0, i, 0)),
            pl.BlockSpec((ROW_BLK, 1), lambda i: (i, 0)),
        ],
        out_shape=[
            jax.ShapeDtypeStruct((C, NPAD, F), jnp.float32),
            jax.ShapeDtypeStruct((NPAD, 1), jnp.float32),
        ],
    )(parts, x)


EPS = E_PAD // NS
EBS = 64
NB = EPS // EBS
NBH2 = NB // 2
CPS = C // NC


@functools.partial(
    pl.kernel,
    out_type=jax.ShapeDtypeStruct((C * NPAD, F), jnp.float32),
    mesh=_mesh,
    scratch_types=[
        pltpu.VMEM((NBH2, EBS), jnp.int32),
        pltpu.VMEM((NBH2, EBS), jnp.int32),
        pltpu.VMEM((2, EBS, F), jnp.float32),
        pltpu.VMEM_SHARED((NPAD, F), jnp.float32),
        pltpu.SemaphoreType.DMA,
        pltpu.SemaphoreType.DMA,
    ],
)
def _scatter_k(xs_hbm, srcadj_hbm, dst_hbm, zeros_hbm, out_hbm,
               src_v, dst_v, rows_v, z_sh, sem0, sem1):
    cid = lax.axis_index("c")
    tid = lax.axis_index("s")

    for cc in range(CPS):
        c = cid * CPS + cc
        pltpu.sync_copy(zeros_hbm, z_sh.at[pl.ds(tid * NROWS_T, NROWS_T)])
        plsc.subcore_barrier()

        for h in range(2):
            pltpu.sync_copy(
                srcadj_hbm.at[pl.ds(c * (E_PAD // EBS) + tid * NB + h * NBH2,
                                    NBH2)],
                src_v)
            pltpu.sync_copy(dst_hbm.at[pl.ds(tid * NB + h * NBH2, NBH2)],
                            dst_v)

            def gather(b, buf, sem):
                return pltpu.make_async_copy(
                    xs_hbm.at[src_v.at[b]], rows_v.at[buf], sem)

            gather(0, 0, sem0).start()

            def body(j, _):
                b0 = 2 * j
                gather(b0 + 1, 1, sem1).start()
                gather(b0, 0, sem0).wait()
                pltpu.sync_copy(rows_v.at[0], z_sh.at[dst_v.at[b0]], add=True)

                @pl.when(j < NBH2 // 2 - 1)
                def _():
                    gather(b0 + 2, 0, sem0).start()

                gather(b0 + 1, 1, sem1).wait()
                pltpu.sync_copy(rows_v.at[1], z_sh.at[dst_v.at[b0 + 1]],
                                add=True)
                return 0

            lax.fori_loop(0, NBH2 // 2, body, 0)

        plsc.subcore_barrier()
        pltpu.sync_copy(
            z_sh.at[pl.ds(tid * NROWS_T, NROWS_T)],
            out_hbm.at[pl.ds(c * NPAD + tid * NROWS_T, NROWS_T)])


def _head_body(z_ref, xs_ref, dinv_ref, x_ref,
               wcat_ref, bcat_ref, wlt_ref, wlb_ref, bl_ref, out_ref):
    y = jnp.concatenate(
        [z_ref[c] + xs_ref[c] for c in range(C)], axis=-1) * dinv_ref[...]
    g = jnp.maximum(jnp.dot(y, wcat_ref[...], preferred_element_type=jnp.float32)
                    + bcat_ref[...], 0.0)
    h = (jnp.dot(g, wlt_ref[...], preferred_element_type=jnp.float32)
         + jnp.dot(x_ref[...], wlb_ref[...], preferred_element_type=jnp.float32)
         + bl_ref[...])
    out_ref[...] = jnp.maximum(h, 0.0)


def _head(z, xs, dinv, x, wcat, bcat, wl_top, wl_bot, bl):
    return pl.pallas_call(
        _head_body,
        grid=(NPAD // ROW_BLK,),
        in_specs=[
            pl.BlockSpec((C, ROW_BLK, F), lambda i: (0, i, 0)),
            pl.BlockSpec((C, ROW_BLK, F), lambda i: (0, i, 0)),
            pl.BlockSpec((ROW_BLK, 1), lambda i: (i, 0)),
            pl.BlockSpec((ROW_BLK, D), lambda i: (i, 0)),
            pl.BlockSpec((D, D), lambda i: (0, 0)),
            pl.BlockSpec((1, D), lambda i: (0, 0)),
            pl.BlockSpec((D, H2), lambda i: (0, 0)),
            pl.BlockSpec((D, H2), lambda i: (0, 0)),
            pl.BlockSpec((1, H2), lambda i: (0, 0)),
        ],
        out_specs=pl.BlockSpec((ROW_BLK, H2), lambda i: (i, 0)),
        out_shape=jax.ShapeDtypeStruct((NPAD, H2), jnp.float32),
    )(z, xs, dinv, x, wcat, bcat.reshape(1, D), wl_top, wl_bot,
      bl.reshape(1, H2))


EB_T = E2_PAD // NW


@functools.partial(
    pl.kernel,
    out_type=jax.ShapeDtypeStruct((E2_PAD,), jnp.float32),
    mesh=_mesh,
    scratch_types=[
        pltpu.VMEM((NB2, EB), jnp.int32),
        pltpu.VMEM((NB2, EB), jnp.int32),
        pltpu.VMEM((2, EB, H2), jnp.float32),
        pltpu.VMEM((2, EB, H2), jnp.float32),
        pltpu.VMEM((EB_T,), jnp.float32),
        pltpu.SemaphoreType.DMA,
        pltpu.SemaphoreType.DMA,
        pltpu.SemaphoreType.DMA,
        pltpu.SemaphoreType.DMA,
    ],
)
def _score_k(h2_hbm, sidx_hbm, didx_hbm, out_hbm,
             sidx_v, didx_v, srows_v, drows_v, out_v, s0, s1, s2, s3):
    wid = lax.axis_index("s") * NC + lax.axis_index("c")
    pltpu.sync_copy(sidx_hbm.at[pl.ds(wid * NB2, NB2)], sidx_v)
    pltpu.sync_copy(didx_hbm.at[pl.ds(wid * NB2, NB2)], didx_v)

    def gs(b, buf, sem):
        return pltpu.make_async_copy(h2_hbm.at[sidx_v.at[b]],
                                     srows_v.at[buf], sem)

    def gd(b, buf, sem):
        return pltpu.make_async_copy(h2_hbm.at[didx_v.at[b]],
                                     drows_v.at[buf], sem)

    gs(0, 0, s0).start()
    gd(0, 0, s1).start()

    lane = lax.iota(jnp.int32, L)

    gdnums = lax.GatherDimensionNumbers(
        offset_dims=(), collapsed_slice_dims=(0,), start_index_map=(0,))

    def hsum(v):
        for s in (1, 2, 4, 8):
            p = lax.gather(v, (lane ^ s)[:, None], gdnums, (1,),
                           mode=lax.GatherScatterMode.PROMISE_IN_BOUNDS)
            v = v + p
        return v

    def dots(b, buf):
        def gbody(g, _):
            res = jnp.zeros((L,), jnp.float32)
            for el in range(L):
                e = g * L + el
                acc = jnp.zeros((L,), jnp.float32)
                for k in range(H2 // L):
                    acc = acc + (srows_v[buf, e, pl.ds(k * L, L)]
                                 * drows_v[buf, e, pl.ds(k * L, L)])
                res = jnp.where(lane == el, hsum(acc), res)
            out_v[pl.ds(b * EB + g * L, L)] = res
            return 0
        lax.fori_loop(0, EB // L, gbody, 0)

    def body(j, _):
        b0 = 2 * j
        gs(b0 + 1, 1, s2).start()
        gd(b0 + 1, 1, s3).start()
        gs(b0, 0, s0).wait()
        gd(b0, 0, s1).wait()
        dots(b0, 0)

        @pl.when(j < NB2 // 2 - 1)
        def _():
            gs(b0 + 2, 0, s0).start()
            gd(b0 + 2, 0, s1).start()

        gs(b0 + 1, 1, s2).wait()
        gd(b0 + 1, 1, s3).wait()
        dots(b0 + 1, 1)
        return 0

    lax.fori_loop(0, NB2 // 2, body, 0)
    pltpu.sync_copy(out_v, out_hbm.at[pl.ds(wid * EB_T, EB_T)])


def kernel(pos_edge_index, neg_edge_index, x, train_pos_edge_index,
           W1, b1, W2, b2, W3, b3, W4, b4, Wl, bl):
    i32 = jnp.int32
    src = train_pos_edge_index[0].astype(i32)
    dst = train_pos_edge_index[1].astype(i32)
    padN = jnp.full((E_PAD - E_TRAIN,), N, i32)
    src_p = jnp.concatenate([src, padN])
    dst_p = jnp.concatenate([dst, padN])
    srcadj = (src_p[None, :] + (jnp.arange(C, dtype=i32) * NPAD)[:, None])
    srcadj = srcadj.reshape(C * (E_PAD // EBS), EBS)
    dst2d = dst_p.reshape(E_PAD // 128, 128)
    dst2d_s = dst_p.reshape(E_PAD // EBS, EBS)
    zeros_rows = jnp.zeros((NROWS_T, F), jnp.float32)

    parts = _hist_k(dst2d).reshape(NC, NPAD, L)
    x_pad = jnp.pad(x, ((0, NPAD - N), (0, 0)))
    xs, dinv = _prep(parts, x_pad)

    z_flat = _scatter_k(xs.reshape(C * NPAD, F), srcadj, dst2d_s, zeros_rows)
    z = z_flat.reshape(C, NPAD, F)

    wcat = jnp.concatenate([W1, W2, W3, W4], axis=1)
    bcat = jnp.concatenate([b1, b2, b3, b4])
    h2 = _head(z, xs, dinv, x_pad, wcat, bcat, Wl[:D], Wl[D:], bl)

    te = jnp.concatenate([pos_edge_index, neg_edge_index], axis=-1).astype(i32)
    pad0 = jnp.zeros((E2_PAD - E_SCORE,), i32)
    te_src = jnp.concatenate([te[0], pad0]).reshape(E2_PAD // EB, EB)
    te_dst = jnp.concatenate([te[1], pad0]).reshape(E2_PAD // EB, EB)
    scores = _score_k(h2, te_src, te_dst)
    return scores[:E_SCORE]


# speedup vs baseline: 4.6937x; 1.0035x over previous
import functools

import jax
import jax.numpy as jnp
from jax import lax
from jax.experimental import pallas as pl
from jax.experimental.pallas import tpu as pltpu
from jax.experimental.pallas import tpu_sc as plsc

N = 10000
NPAD = 10240
D = 1024
H2 = 512
F = 128
C = D // F
NC, NS, L = 2, 16, 16
NW = NC * NS
E_TRAIN = 160000
E_PAD = 163840
E_SCORE = 100000
EB = 32
NB2 = 104
E2_PAD = 32 * EB * NB2

ROW_BLK = 640

_mesh = plsc.VectorSubcoreMesh(core_axis_name="c", subcore_axis_name="s")


NBH = E_PAD // (NC * NS * 128)
NROWS_T = NPAD // NS


@functools.partial(
    pl.kernel,
    out_type=jax.ShapeDtypeStruct((NC * NPAD, L), jnp.float32),
    mesh=_mesh,
    scratch_types=[
        pltpu.VMEM((NBH, 128), jnp.int32),
        pltpu.VMEM((128, L), jnp.float32),
        pltpu.VMEM((128, L), jnp.float32),
        pltpu.VMEM_SHARED((NPAD, L), jnp.float32),
    ],
)
def _hist_k(dst_hbm, out_hbm, didx_v, ones_v, zero_v, deg_sh):
    cid = lax.axis_index("c")
    tid = lax.axis_index("s")
    pltpu.sync_copy(dst_hbm.at[pl.ds((cid * NS + tid) * NBH, NBH)], didx_v)
    ones16 = jnp.ones((L,), jnp.float32)
    zero16 = jnp.zeros((L,), jnp.float32)

    def fill(i, _):
        ones_v[i, :] = ones16
        zero_v[i, :] = zero16
        return 0

    lax.fori_loop(0, 128, fill, 0)
    for r in range(NROWS_T // 128):
        pltpu.sync_copy(zero_v, deg_sh.at[pl.ds(tid * NROWS_T + r * 128, 128)])
    plsc.subcore_barrier()

    def body(b, _):
        pltpu.sync_copy(ones_v, deg_sh.at[didx_v.at[b]], add=True)
        return 0

    lax.fori_loop(0, NBH, body, 0)
    plsc.subcore_barrier()
    pltpu.sync_copy(deg_sh.at[pl.ds(tid * NROWS_T, NROWS_T)],
                    out_hbm.at[pl.ds(cid * NPAD + tid * NROWS_T, NROWS_T)])


def _prep_body(parts_ref, x_ref, xs_ref, dinv_ref):
    p = parts_ref[:, :, 0:1]
    deg = p[0] + p[1] + 1.0
    dinv = lax.rsqrt(deg)
    xsb = x_ref[...] * dinv
    for c in range(C):
        xs_ref[c] = xsb[:, c * F:(c + 1) * F]
    dinv_ref[...] = dinv


def _prep(parts, x):
    return pl.pallas_call(
        _prep_body,
        grid=(NPAD // ROW_BLK,),
        in_specs=[
            pl.BlockSpec((NC, ROW_BLK, L), lambda i: (0, i, 0)),
            pl.BlockSpec((ROW_BLK, D), lambda i: (i, 0)),
        ],
        out_specs=[
            pl.BlockSpec((C, ROW_BLK, F), lambda i: (0, i, 0)),
            pl.BlockSpec((ROW_BLK, 1), lambda i: (i, 0)),
        ],
        out_shape=[
            jax.ShapeDtypeStruct((C, NPAD, F), jnp.float32),
            jax.ShapeDtypeStruct((NPAD, 1), jnp.float32),
        ],
    )(parts, x)


EPS = E_PAD // NS
EBS = 64
NB = EPS // EBS
NB8 = 16
NBUF = 3


@functools.partial(
    pl.kernel,
    out_type=jax.ShapeDtypeStruct((C * NPAD, F), jnp.float32),
    mesh=_mesh,
    scratch_types=[
        pltpu.VMEM((NB8, EBS), jnp.int32),
        pltpu.VMEM((NB8, EBS), jnp.int32),
        pltpu.VMEM((NBUF, EBS, F), jnp.float32),
        pltpu.VMEM_SHARED((NPAD, F), jnp.float32),
        pltpu.SemaphoreType.DMA,
        pltpu.SemaphoreType.DMA,
        pltpu.SemaphoreType.DMA,
    ],
)
def _scatter_k(xs_hbm, srcadj_hbm, dst_hbm, zeros_hbm, out_hbm,
               src_v, dst_v, rows_v, z_sh, g0, g1, g2):
    cid = lax.axis_index("c")
    tid = lax.axis_index("s")
    gs = (g0, g1, g2)
    CPS = C // NC

    for cc in range(CPS):
        c = cid * CPS + cc
        pltpu.sync_copy(zeros_hbm, z_sh.at[pl.ds(tid * NROWS_T, NROWS_T)])
        plsc.subcore_barrier()

        def block(h, _):
            pltpu.sync_copy(
                srcadj_hbm.at[pl.ds(c * (E_PAD // EBS) + tid * NB + h * NB8,
                                    NB8)],
                src_v)
            pltpu.sync_copy(dst_hbm.at[pl.ds(tid * NB + h * NB8, NB8)], dst_v)

            def wait_gather(b, buf):
                pltpu.make_async_copy(xs_hbm.at[src_v.at[b]],
                                      rows_v.at[buf], gs[buf]).wait()

            for b in range(NBUF):
                pltpu.async_copy(xs_hbm.at[src_v.at[b]],
                                 rows_v.at[b], gs[b])
            for b in range(NB8):
                buf = b % NBUF
                wait_gather(b, buf)
                pltpu.sync_copy(rows_v.at[buf], z_sh.at[dst_v.at[b]],
                                add=True)
                if b + NBUF < NB8:
                    pltpu.async_copy(xs_hbm.at[src_v.at[b + NBUF]],
                                     rows_v.at[buf], gs[buf])
            return 0

        lax.fori_loop(0, NB // NB8, block, 0)
        plsc.subcore_barrier()
        pltpu.sync_copy(
            z_sh.at[pl.ds(tid * NROWS_T, NROWS_T)],
            out_hbm.at[pl.ds(c * NPAD + tid * NROWS_T, NROWS_T)])


def _head_body(z_ref, xs_ref, dinv_ref, x_ref,
               wcat_ref, bcat_ref, wlt_ref, wlb_ref, bl_ref, out_ref):
    y = jnp.concatenate(
        [z_ref[c] + xs_ref[c] for c in range(C)], axis=-1) * dinv_ref[...]
    g = jnp.maximum(jnp.dot(y, wcat_ref[...], preferred_element_type=jnp.float32)
                    + bcat_ref[...], 0.0)
    h = (jnp.dot(g, wlt_ref[...], preferred_element_type=jnp.float32)
         + jnp.dot(x_ref[...], wlb_ref[...], preferred_element_type=jnp.float32)
         + bl_ref[...])
    out_ref[...] = jnp.maximum(h, 0.0)


def _head(z, xs, dinv, x, wcat, bcat, wl_top, wl_bot, bl):
    return pl.pallas_call(
        _head_body,
        grid=(NPAD // ROW_BLK,),
        in_specs=[
            pl.BlockSpec((C, ROW_BLK, F), lambda i: (0, i, 0)),
            pl.BlockSpec((C, ROW_BLK, F), lambda i: (0, i, 0)),
            pl.BlockSpec((ROW_BLK, 1), lambda i: (i, 0)),
            pl.BlockSpec((ROW_BLK, D), lambda i: (i, 0)),
            pl.BlockSpec((D, D), lambda i: (0, 0)),
            pl.BlockSpec((1, D), lambda i: (0, 0)),
            pl.BlockSpec((D, H2), lambda i: (0, 0)),
            pl.BlockSpec((D, H2), lambda i: (0, 0)),
            pl.BlockSpec((1, H2), lambda i: (0, 0)),
        ],
        out_specs=pl.BlockSpec((ROW_BLK, H2), lambda i: (i, 0)),
        out_shape=jax.ShapeDtypeStruct((NPAD, H2), jnp.float32),
    )(z, xs, dinv, x, wcat, bcat.reshape(1, D), wl_top, wl_bot,
      bl.reshape(1, H2))


EB_T = E2_PAD // NW


@functools.partial(
    pl.kernel,
    out_type=jax.ShapeDtypeStruct((E2_PAD,), jnp.float32),
    mesh=_mesh,
    scratch_types=[
        pltpu.VMEM((NB2, EB), jnp.int32),
        pltpu.VMEM((NB2, EB), jnp.int32),
        pltpu.VMEM((2, EB, H2), jnp.float32),
        pltpu.VMEM((2, EB, H2), jnp.float32),
        pltpu.VMEM((EB_T,), jnp.float32),
        pltpu.SemaphoreType.DMA,
        pltpu.SemaphoreType.DMA,
        pltpu.SemaphoreType.DMA,
        pltpu.SemaphoreType.DMA,
    ],
)
def _score_k(h2_hbm, sidx_hbm, didx_hbm, out_hbm,
             sidx_v, didx_v, srows_v, drows_v, out_v, s0, s1, s2, s3):
    wid = lax.axis_index("s") * NC + lax.axis_index("c")
    pltpu.sync_copy(sidx_hbm.at[pl.ds(wid * NB2, NB2)], sidx_v)
    pltpu.sync_copy(didx_hbm.at[pl.ds(wid * NB2, NB2)], didx_v)

    def gs(b, buf, sem):
        return pltpu.make_async_copy(h2_hbm.at[sidx_v.at[b]],
                                     srows_v.at[buf], sem)

    def gd(b, buf, sem):
        return pltpu.make_async_copy(h2_hbm.at[didx_v.at[b]],
                                     drows_v.at[buf], sem)

    gs(0, 0, s0).start()
    gd(0, 0, s1).start()

    lane = lax.iota(jnp.int32, L)

    gdnums = lax.GatherDimensionNumbers(
        offset_dims=(), collapsed_slice_dims=(0,), start_index_map=(0,))

    def hsum(v):
        for s in (1, 2, 4, 8):
            p = lax.gather(v, (lane ^ s)[:, None], gdnums, (1,),
                           mode=lax.GatherScatterMode.PROMISE_IN_BOUNDS)
            v = v + p
        return v

    def dots(b, buf):
        def gbody(g, _):
            res = jnp.zeros((L,), jnp.float32)
            for el in range(L):
                e = g * L + el
                acc = jnp.zeros((L,), jnp.float32)
                for k in range(H2 // L):
                    acc = acc + (srows_v[buf, e, pl.ds(k * L, L)]
                                 * drows_v[buf, e, pl.ds(k * L, L)])
                res = jnp.where(lane == el, hsum(acc), res)
            out_v[pl.ds(b * EB + g * L, L)] = res
            return 0
        lax.fori_loop(0, EB // L, gbody, 0)

    def body(j, _):
        b0 = 2 * j
        gs(b0 + 1, 1, s2).start()
        gd(b0 + 1, 1, s3).start()
        gs(b0, 0, s0).wait()
        gd(b0, 0, s1).wait()
        dots(b0, 0)

        @pl.when(j < NB2 // 2 - 1)
        def _():
            gs(b0 + 2, 0, s0).start()
            gd(b0 + 2, 0, s1).start()

        gs(b0 + 1, 1, s2).wait()
        gd(b0 + 1, 1, s3).wait()
        dots(b0 + 1, 1)
        return 0

    lax.fori_loop(0, NB2 // 2, body, 0)
    pltpu.sync_copy(out_v, out_hbm.at[pl.ds(wid * EB_T, EB_T)])


def kernel(pos_edge_index, neg_edge_index, x, train_pos_edge_index,
           W1, b1, W2, b2, W3, b3, W4, b4, Wl, bl):
    i32 = jnp.int32
    src = train_pos_edge_index[0].astype(i32)
    dst = train_pos_edge_index[1].astype(i32)
    padN = jnp.full((E_PAD - E_TRAIN,), N, i32)
    src_p = jnp.concatenate([src, padN])
    dst_p = jnp.concatenate([dst, padN])
    srcadj = (src_p[None, :] + (jnp.arange(C, dtype=i32) * NPAD)[:, None])
    srcadj = srcadj.reshape(C * (E_PAD // EBS), EBS)
    dst2d = dst_p.reshape(E_PAD // 128, 128)
    dst2d_s = dst_p.reshape(E_PAD // EBS, EBS)
    zeros_rows = jnp.zeros((NROWS_T, F), jnp.float32)

    parts = _hist_k(dst2d).reshape(NC, NPAD, L)
    x_pad = jnp.pad(x, ((0, NPAD - N), (0, 0)))
    xs, dinv = _prep(parts, x_pad)

    z_flat = _scatter_k(xs.reshape(C * NPAD, F), srcadj, dst2d_s, zeros_rows)
    z = z_flat.reshape(C, NPAD, F)

    wcat = jnp.concatenate([W1, W2, W3, W4], axis=1)
    bcat = jnp.concatenate([b1, b2, b3, b4])
    h2 = _head(z, xs, dinv, x_pad, wcat, bcat, Wl[:D], Wl[D:], bl)

    te = jnp.concatenate([pos_edge_index, neg_edge_index], axis=-1).astype(i32)
    pad0 = jnp.zeros((E2_PAD - E_SCORE,), i32)
    te_src = jnp.concatenate([te[0], pad0]).reshape(E2_PAD // EB, EB)
    te_dst = jnp.concatenate([te[1], pad0]).reshape(E2_PAD // EB, EB)
    scores = _score_k(h2, te_src, te_dst)
    return scores[:E_SCORE]
